# baseline (speedup 1.0000x reference)
import functools

import jax
import jax.numpy as jnp
from jax.experimental import pallas as pl
from jax.experimental.pallas import tpu as pltpu

_PRE_TOP_K = 1000
_KEEP_TOP_K = 100
_IOU_THR = 0.65
_SCORE_THR = 0.25
_N = 20000
_C = 80
_BN = 2000


def _dense_body(cls_ref, bx_ref, by_ref, bw_ref, bh_ref,
                p0_ref, p1_ref, p2_ref, p3_ref, st_ref,
                smax_ref, lab_ref, x1_ref, y1_ref, x2_ref, y2_ref):
    s = jax.nn.sigmoid(cls_ref[...])
    m = jnp.max(s, axis=1, keepdims=True)
    smax_ref[...] = m
    iota = jax.lax.broadcasted_iota(jnp.int32, s.shape, 1).astype(jnp.float32)
    lab = jnp.min(jnp.where(s == m, iota, jnp.float32(_C)), axis=1, keepdims=True)
    lab_ref[...] = lab

    stride = st_ref[...]
    xc = (p0_ref[...] + p2_ref[...]) * 0.5
    yc = (p1_ref[...] + p3_ref[...]) * 0.5
    w = p2_ref[...] - p0_ref[...]
    h = p3_ref[...] - p1_ref[...]
    sx = jax.nn.sigmoid(bx_ref[...])
    sy = jax.nn.sigmoid(by_ref[...])
    sw = jax.nn.sigmoid(bw_ref[...])
    sh = jax.nn.sigmoid(bh_ref[...])
    xcp = (sx - 0.5) * 2.0 * stride + xc
    ycp = (sy - 0.5) * 2.0 * stride + yc
    wp = (sw * 2.0) ** 2 * w
    hp = (sh * 2.0) ** 2 * h
    x1_ref[...] = xcp - wp * 0.5
    y1_ref[...] = ycp - hp * 0.5
    x2_ref[...] = xcp + wp * 0.5
    y2_ref[...] = ycp + hp * 0.5


def _dense_stage(cls2d, bcols, pcols, stcol):
    n = cls2d.shape[0]
    grid = n // _BN
    col_spec = pl.BlockSpec((_BN, 1), lambda i: (i, 0))
    out_sds = jax.ShapeDtypeStruct((n, 1), jnp.float32)
    return pl.pallas_call(
        _dense_body,
        grid=(grid,),
        in_specs=[pl.BlockSpec((_BN, _C), lambda i: (i, 0))] + [col_spec] * 9,
        out_specs=[col_spec] * 6,
        out_shape=[out_sds] * 6,
    )(cls2d, *bcols, *pcols, stcol)


def _pairwise_iou(box, boxes):
    x1 = jnp.maximum(box[0], boxes[:, 0])
    y1 = jnp.maximum(box[1], boxes[:, 1])
    x2 = jnp.minimum(box[2], boxes[:, 2])
    y2 = jnp.minimum(box[3], boxes[:, 3])
    inter = jnp.clip(x2 - x1, 0.0) * jnp.clip(y2 - y1, 0.0)
    a1 = (box[2] - box[0]) * (box[3] - box[1])
    a2 = (boxes[:, 2] - boxes[:, 0]) * (boxes[:, 3] - boxes[:, 1])
    return inter / (a1 + a2 - inter + 1e-7)


def _nms_tail(boxes, scores, labels):
    top_scores, idx = jax.lax.top_k(scores, _PRE_TOP_K)
    top_boxes = boxes[idx]
    top_labels = labels[idx]
    arange = jnp.arange(_PRE_TOP_K)

    def body(i, supp):
        valid = jnp.logical_not(supp[i]) & (top_scores[i] > _SCORE_THR)
        ious = _pairwise_iou(top_boxes[i], top_boxes)
        return supp | (valid & (ious > _IOU_THR) & (arange > i))

    supp = jax.lax.fori_loop(0, _PRE_TOP_K, body, jnp.zeros((_PRE_TOP_K,), dtype=bool))
    keep = jnp.logical_not(supp) & (top_scores > _SCORE_THR)
    masked = jnp.where(keep, top_scores, -1.0)
    fin_scores, fin_idx = jax.lax.top_k(masked, _KEEP_TOP_K)
    fin_boxes = top_boxes[fin_idx]
    fin_labels = top_labels[fin_idx]
    return jnp.concatenate(
        [fin_boxes, fin_scores[:, None], fin_labels[:, None]], axis=-1)


def kernel(cls_scores, bbox_preds, priors, strides):
    cls2d = cls_scores.reshape(_N, _C)
    bcols = [bbox_preds[0, :, k:k + 1] for k in range(4)]
    pcols = [priors[:, k:k + 1] for k in range(4)]
    stcol = strides.reshape(_N, 1)
    smax, lab, x1, y1, x2, y2 = _dense_stage(cls2d, bcols, pcols, stcol)
    boxes = jnp.concatenate([x1, y1, x2, y2], axis=1)
    out = _nms_tail(boxes, smax[:, 0], lab[:, 0])
    return out[None]


# speedup vs baseline: 26.3608x; 26.3608x over previous
import functools

import jax
import jax.numpy as jnp
from jax.experimental import pallas as pl
from jax.experimental.pallas import tpu as pltpu

_PRE_TOP_K = 1000
_KEEP_TOP_K = 100
_IOU_THR = 0.65
_SCORE_THR = 0.25
_N = 20000
_C = 80
_BN = 2000


def _dense_body(cls_ref, bx_ref, by_ref, bw_ref, bh_ref,
                p0_ref, p1_ref, p2_ref, p3_ref, st_ref,
                smax_ref, lab_ref, x1_ref, y1_ref, x2_ref, y2_ref):
    s = jax.nn.sigmoid(cls_ref[...])
    m = jnp.max(s, axis=1, keepdims=True)
    smax_ref[...] = m
    iota = jax.lax.broadcasted_iota(jnp.int32, s.shape, 1).astype(jnp.float32)
    lab = jnp.min(jnp.where(s == m, iota, jnp.float32(_C)), axis=1, keepdims=True)
    lab_ref[...] = lab

    stride = st_ref[...]
    xc = (p0_ref[...] + p2_ref[...]) * 0.5
    yc = (p1_ref[...] + p3_ref[...]) * 0.5
    w = p2_ref[...] - p0_ref[...]
    h = p3_ref[...] - p1_ref[...]
    sx = jax.nn.sigmoid(bx_ref[...])
    sy = jax.nn.sigmoid(by_ref[...])
    sw = jax.nn.sigmoid(bw_ref[...])
    sh = jax.nn.sigmoid(bh_ref[...])
    xcp = (sx - 0.5) * 2.0 * stride + xc
    ycp = (sy - 0.5) * 2.0 * stride + yc
    wp = (sw * 2.0) ** 2 * w
    hp = (sh * 2.0) ** 2 * h
    x1_ref[...] = xcp - wp * 0.5
    y1_ref[...] = ycp - hp * 0.5
    x2_ref[...] = xcp + wp * 0.5
    y2_ref[...] = ycp + hp * 0.5


def _dense_stage(cls2d, bcols, pcols, stcol):
    n = cls2d.shape[0]
    grid = n // _BN
    col_spec = pl.BlockSpec((_BN, 1), lambda i: (i, 0))
    out_sds = jax.ShapeDtypeStruct((n, 1), jnp.float32)
    return pl.pallas_call(
        _dense_body,
        grid=(grid,),
        in_specs=[pl.BlockSpec((_BN, _C), lambda i: (i, 0))] + [col_spec] * 9,
        out_specs=[col_spec] * 6,
        out_shape=[out_sds] * 6,
    )(cls2d, *bcols, *pcols, stcol)


_TOPP = 1024


def _nms_body(x1r, y1r, x2r, y2r, sr, x1c, y1c, x2c, y2c,
              supp_out, m_ref):
    ix1 = jnp.maximum(x1r[...], x1c[...])
    iy1 = jnp.maximum(y1r[...], y1c[...])
    ix2 = jnp.minimum(x2r[...], x2c[...])
    iy2 = jnp.minimum(y2r[...], y2c[...])
    inter = jnp.clip(ix2 - ix1, 0.0) * jnp.clip(iy2 - iy1, 0.0)
    ar = (x2r[...] - x1r[...]) * (y2r[...] - y1r[...])
    ac = (x2c[...] - x1c[...]) * (y2c[...] - y1c[...])
    iou = inter / (ar + ac - inter + 1e-7)
    ii = jax.lax.broadcasted_iota(jnp.int32, (_TOPP, _TOPP), 0)
    jj = jax.lax.broadcasted_iota(jnp.int32, (_TOPP, _TOPP), 1)
    m = ((iou > _IOU_THR) & (jj > ii) & (sr[...] > _SCORE_THR))
    m_ref[...] = m.astype(jnp.float32)

    lane = jax.lax.broadcasted_iota(jnp.int32, (1, _TOPP), 1)

    def body(i, supp):
        mrow = m_ref[pl.ds(i, 1), :]
        onehot = (lane == i).astype(jnp.float32)
        si = jnp.sum(supp * onehot)
        return jnp.maximum(supp, mrow * (1.0 - si))

    supp = jax.lax.fori_loop(0, _PRE_TOP_K,
                             body, jnp.zeros((1, _TOPP), jnp.float32))
    supp_out[...] = supp


def _nms_stage(tb, ts):
    rows = [tb[:, k:k + 1] for k in range(4)] + [ts[:, None]]
    cols = [tb[:, k].reshape(1, _TOPP) for k in range(4)]
    supp = pl.pallas_call(
        _nms_body,
        in_specs=[pl.BlockSpec((_TOPP, 1), lambda: (0, 0))] * 5
        + [pl.BlockSpec((1, _TOPP), lambda: (0, 0))] * 4,
        out_specs=pl.BlockSpec((1, _TOPP), lambda: (0, 0)),
        out_shape=jax.ShapeDtypeStruct((1, _TOPP), jnp.float32),
        scratch_shapes=[pltpu.VMEM((_TOPP, _TOPP), jnp.float32)],
    )(*rows, *cols)
    return supp[0]


def _pairwise_iou(box, boxes):
    x1 = jnp.maximum(box[0], boxes[:, 0])
    y1 = jnp.maximum(box[1], boxes[:, 1])
    x2 = jnp.minimum(box[2], boxes[:, 2])
    y2 = jnp.minimum(box[3], boxes[:, 3])
    inter = jnp.clip(x2 - x1, 0.0) * jnp.clip(y2 - y1, 0.0)
    a1 = (box[2] - box[0]) * (box[3] - box[1])
    a2 = (boxes[:, 2] - boxes[:, 0]) * (boxes[:, 3] - boxes[:, 1])
    return inter / (a1 + a2 - inter + 1e-7)


def _nms_tail(boxes, scores, labels):
    top_scores, idx = jax.lax.top_k(scores, _PRE_TOP_K)
    top_boxes = boxes[idx]
    top_labels = labels[idx]
    tb = jnp.concatenate(
        [top_boxes, jnp.zeros((_TOPP - _PRE_TOP_K, 4), jnp.float32)], axis=0)
    ts = jnp.concatenate(
        [top_scores, jnp.full((_TOPP - _PRE_TOP_K,), -1e30, jnp.float32)], axis=0)
    supp = _nms_stage(tb, ts)[:_PRE_TOP_K] > 0.0
    keep = jnp.logical_not(supp) & (top_scores > _SCORE_THR)
    masked = jnp.where(keep, top_scores, -1.0)
    fin_scores, fin_idx = jax.lax.top_k(masked, _KEEP_TOP_K)
    fin_boxes = top_boxes[fin_idx]
    fin_labels = top_labels[fin_idx]
    return jnp.concatenate(
        [fin_boxes, fin_scores[:, None], fin_labels[:, None]], axis=-1)


def kernel(cls_scores, bbox_preds, priors, strides):
    cls2d = cls_scores.reshape(_N, _C)
    bcols = [bbox_preds[0, :, k:k + 1] for k in range(4)]
    pcols = [priors[:, k:k + 1] for k in range(4)]
    stcol = strides.reshape(_N, 1)
    smax, lab, x1, y1, x2, y2 = _dense_stage(cls2d, bcols, pcols, stcol)
    boxes = jnp.concatenate([x1, y1, x2, y2], axis=1)
    out = _nms_tail(boxes, smax[:, 0], lab[:, 0])
    return out[None]
